# baseline (speedup 1.0000x reference)
import functools

import jax
import jax.numpy as jnp
from jax.experimental import pallas as pl
from jax.experimental.pallas import tpu as pltpu


def _make_kernel(nstream, h):
    def _gcn_kernel(*refs):
        a_refs = refs[:nstream]
        x_ref, w_ref, o_ref, y_ref = refs[nstream:]

        @pl.when(pl.program_id(0) == 0)
        def _compute_y():
            x = x_ref[...].astype(jnp.bfloat16)
            w = w_ref[...].astype(jnp.bfloat16)
            y_ref[...] = jnp.dot(
                x, w, preferred_element_type=jnp.float32
            ).astype(jnp.bfloat16)

        y = y_ref[...]
        for j in range(nstream):
            a = a_refs[j][...].astype(jnp.bfloat16)
            o_ref[j * h : (j + 1) * h, :] = jnp.dot(
                a, y, preferred_element_type=jnp.float32
            )

    return _gcn_kernel


@functools.partial(jax.jit, static_argnames=("h", "nstream"))
def _one_support(x, adj, w, h, nstream):
    n, in_f = x.shape
    out_f = w.shape[1]
    group = h * nstream
    num_groups = n // group
    a_specs = [
        pl.BlockSpec((h, n), functools.partial(lambda j, m: (m * nstream + j, 0), j))
        for j in range(nstream)
    ]
    return pl.pallas_call(
        _make_kernel(nstream, h),
        grid=(num_groups,),
        in_specs=a_specs
        + [
            pl.BlockSpec((n, in_f), lambda m: (0, 0)),
            pl.BlockSpec((in_f, out_f), lambda m: (0, 0)),
        ],
        out_specs=pl.BlockSpec((group, out_f), lambda m: (m, 0)),
        out_shape=jax.ShapeDtypeStruct((n, out_f), jnp.float32),
        scratch_shapes=[pltpu.VMEM((n, out_f), jnp.bfloat16)],
        compiler_params=pltpu.CompilerParams(
            dimension_semantics=("arbitrary",),
        ),
    )(*([adj] * nstream), x, w)


def kernel(input, support, W):
    x = input
    out = None
    for i in range(support.shape[0]):
        o = _one_support(x, support[i], W[i], h=80, nstream=5)
        out = o if out is None else out + o
    return out


# speedup vs baseline: 1.0136x; 1.0136x over previous
import functools

import jax
import jax.numpy as jnp
from jax.experimental import pallas as pl
from jax.experimental.pallas import tpu as pltpu


def _make_kernel(nstream, h):
    def _gcn_kernel(*refs):
        a_refs = refs[:nstream]
        x_ref, w_ref, o_ref, y_ref = refs[nstream:]

        @pl.when(pl.program_id(0) == 0)
        def _compute_y():
            x = x_ref[...].astype(jnp.bfloat16)
            w = w_ref[...].astype(jnp.bfloat16)
            y_ref[...] = jnp.dot(
                x, w, preferred_element_type=jnp.float32
            ).astype(jnp.bfloat16)

        y = y_ref[...]
        for j in range(nstream):
            a = a_refs[j][...].astype(jnp.bfloat16)
            o_ref[j * h : (j + 1) * h, :] = jnp.dot(
                a, y, preferred_element_type=jnp.float32
            )

    return _gcn_kernel


@functools.partial(jax.jit, static_argnames=("h", "nstream"))
def _one_support(x, adj, w, h, nstream):
    n, in_f = x.shape
    out_f = w.shape[1]
    group = h * nstream
    num_groups = n // group
    a_specs = [
        pl.BlockSpec((h, n), functools.partial(lambda j, m: (m * nstream + j, 0), j))
        for j in range(nstream)
    ]
    return pl.pallas_call(
        _make_kernel(nstream, h),
        grid=(num_groups,),
        in_specs=a_specs
        + [
            pl.BlockSpec((n, in_f), lambda m: (0, 0)),
            pl.BlockSpec((in_f, out_f), lambda m: (0, 0)),
        ],
        out_specs=pl.BlockSpec((group, out_f), lambda m: (m, 0)),
        out_shape=jax.ShapeDtypeStruct((n, out_f), jnp.float32),
        scratch_shapes=[pltpu.VMEM((n, out_f), jnp.bfloat16)],
        compiler_params=pltpu.CompilerParams(
            dimension_semantics=("arbitrary",),
        ),
    )(*([adj] * nstream), x, w)


def kernel(input, support, W):
    x = input
    out = None
    for i in range(support.shape[0]):
        o = _one_support(x, support[i], W[i], h=200, nstream=2)
        out = o if out is None else out + o
    return out
